# speedup vs baseline: 1.2739x; 1.2739x over previous
import jax
import jax.numpy as jnp
from jax.experimental import pallas as pl
from jax.experimental.pallas import tpu as pltpu


def _conv_kernel(x_ref, wc_ref, wl_ref, b_ref, o_ref):
    h, wo, c2 = x_ref.shape
    c = c2 // 2
    ho = h // 2
    m = ho * wo
    cout = o_ref.shape[0]

    xv = x_ref[...]

    xsh = jnp.concatenate(
        [jnp.zeros((h, 1, c), xv.dtype), xv[:, : wo - 1, c:]], axis=1)

    xp = xv.reshape(ho, 2, wo, c2)
    r_even = xp[:, 0]
    r_odd = xp[:, 1]
    sp = xsh.reshape(ho, 2, wo, c)
    s_even = sp[:, 0]
    s_odd = sp[:, 1]

    zrow2 = jnp.zeros((1, wo, c2), xv.dtype)
    zrow1 = jnp.zeros((1, wo, c), xv.dtype)
    taps_c = (
        jnp.concatenate([zrow2, r_odd[: ho - 1]], axis=0),
        r_even,
        r_odd,
    )
    taps_l = (
        jnp.concatenate([zrow1, s_odd[: ho - 1]], axis=0),
        s_even,
        s_odd,
    )

    acc = jnp.zeros((cout, m), jnp.float32)
    for ki in range(3):
        a2 = taps_c[ki].reshape(m, c2)
        acc += jax.lax.dot_general(
            wc_ref[ki], a2, (((0,), (1,)), ((), ())),
            preferred_element_type=jnp.float32)
        l2 = taps_l[ki].reshape(m, c)
        acc += jax.lax.dot_general(
            wl_ref[ki], l2, (((0,), (1,)), ((), ())),
            preferred_element_type=jnp.float32)

    o_ref[...] = acc + b_ref[:, 0:1]


def kernel(x, weight, bias):
    n, c, h, w = x.shape
    cout = weight.shape[0]
    ho, wo = h // 2, w // 2
    m = ho * wo

    xt = jnp.transpose(x, (0, 2, 3, 1)).astype(jnp.bfloat16)
    xm = xt.reshape(n, h, wo, 2 * c)

    wt = jnp.transpose(weight, (2, 3, 1, 0)).astype(jnp.bfloat16)
    w_center = jnp.concatenate([wt[:, 1], wt[:, 2]], axis=1)
    w_left = wt[:, 0]
    b2 = jnp.broadcast_to(bias.astype(jnp.float32)[:, None], (cout, 128))

    out = pl.pallas_call(
        _conv_kernel,
        out_shape=jax.ShapeDtypeStruct((n, cout, m), jnp.float32),
        grid=(n,),
        in_specs=[
            pl.BlockSpec((None, h, wo, 2 * c), lambda i: (i, 0, 0, 0)),
            pl.BlockSpec((3, 2 * c, cout), lambda i: (0, 0, 0)),
            pl.BlockSpec((3, c, cout), lambda i: (0, 0, 0)),
            pl.BlockSpec((cout, 128), lambda i: (0, 0)),
        ],
        out_specs=pl.BlockSpec((None, cout, m), lambda i: (i, 0, 0)),
        compiler_params=pltpu.CompilerParams(
            dimension_semantics=("parallel",)),
    )(xm, w_center, w_left, b2)

    return out.reshape(n, cout, ho, wo)


# speedup vs baseline: 1.4333x; 1.1251x over previous
import functools

import jax
import jax.numpy as jnp
from jax.experimental import pallas as pl
from jax.experimental.pallas import tpu as pltpu


def _conv_kernel(x_ref, w_ref, b_ref, o_ref, *, h):
    c, hw = x_ref.shape
    cout = o_ref.shape[0]
    m = o_ref.shape[1]
    w = hw // h
    ho, wo = h // 2, w // 2

    xb = x_ref[...].astype(jnp.bfloat16)
    xt = jnp.transpose(xb)
    xi = pltpu.bitcast(xt, jnp.int32)
    half0 = pltpu.unpack_elementwise(
        xi, index=0, packed_dtype=jnp.bfloat16,
        unpacked_dtype=jnp.float32).astype(jnp.bfloat16)
    half1 = pltpu.unpack_elementwise(
        xi, index=1, packed_dtype=jnp.bfloat16,
        unpacked_dtype=jnp.float32).astype(jnp.bfloat16)

    even = half0.reshape(h, wo, c)
    odd = half1.reshape(h, wo, c)

    lft = jnp.concatenate(
        [jnp.zeros((h, 1, c), jnp.bfloat16), odd[:, : wo - 1, :]], axis=1)
    col_taps = (lft, even, odd)

    acc = jnp.zeros((cout, m), jnp.float32)
    for kj in range(3):
        cp = col_taps[kj].reshape(ho, 2, wo, c)
        r_even = cp[:, 0]
        r_odd = cp[:, 1]
        row_taps = (
            jnp.concatenate(
                [jnp.zeros((1, wo, c), jnp.bfloat16), r_odd[: ho - 1]],
                axis=0),
            r_even,
            r_odd,
        )
        for ki in range(3):
            a2 = row_taps[ki].reshape(m, c)
            acc += jax.lax.dot_general(
                w_ref[ki, kj], a2, (((0,), (1,)), ((), ())),
                preferred_element_type=jnp.float32)

    o_ref[...] = acc + b_ref[:, 0:1]


def kernel(x, weight, bias):
    n, c, h, w = x.shape
    cout = weight.shape[0]
    ho, wo = h // 2, w // 2
    m = ho * wo

    x2 = x.reshape(n, c, h * w)
    w9 = jnp.transpose(weight, (2, 3, 1, 0)).astype(jnp.bfloat16)
    b2 = jnp.broadcast_to(bias.astype(jnp.float32)[:, None], (cout, 128))

    out = pl.pallas_call(
        functools.partial(_conv_kernel, h=h),
        out_shape=jax.ShapeDtypeStruct((n, cout, m), jnp.float32),
        grid=(n,),
        in_specs=[
            pl.BlockSpec((None, c, h * w), lambda i: (i, 0, 0)),
            pl.BlockSpec((3, 3, c, cout), lambda i: (0, 0, 0, 0)),
            pl.BlockSpec((cout, 128), lambda i: (0, 0)),
        ],
        out_specs=pl.BlockSpec((None, cout, m), lambda i: (i, 0, 0)),
        compiler_params=pltpu.CompilerParams(
            dimension_semantics=("parallel",)),
    )(x2, w9, b2)

    return out.reshape(n, cout, ho, wo)
